# baseline (speedup 1.0000x reference)
import functools
import math

import jax
import jax.numpy as jnp
from jax import lax
from jax.experimental import pallas as pl
from jax.experimental.pallas import tpu as pltpu
from jax.experimental.pallas import tpu_sc as plsc

_SCALE = math.sqrt(512.0)
_CHUNK = 64


def _scale_body(t_ref, o_ref):
    o_ref[...] = t_ref[...] * _SCALE


def _scale_table(table):
    return pl.pallas_call(
        _scale_body,
        out_shape=jax.ShapeDtypeStruct(table.shape, table.dtype),
    )(table)


@functools.cache
def _make_gather(B, V, D):
    info = plsc.get_sparse_core_info()
    nc, ns = info.num_cores, info.num_subcores
    nw = nc * ns
    assert B % (nw * _CHUNK) == 0
    b_per_w = B // nw
    n_chunks = b_per_w // _CHUNK
    mesh = plsc.VectorSubcoreMesh(core_axis_name="c", subcore_axis_name="s")

    @functools.partial(
        pl.kernel,
        out_type=jax.ShapeDtypeStruct((B, D), jnp.float32),
        mesh=mesh,
        scratch_types=[
            pltpu.VMEM((_CHUNK,), jnp.int32),
            pltpu.VMEM((_CHUNK, D), jnp.float32),
            pltpu.SemaphoreType.DMA,
        ],
        compiler_params=pltpu.CompilerParams(use_tc_tiling_on_sc=False),
    )
    def gather(idx_hbm, tbl_hbm, out_hbm, idx_v, rows_v, sem):
        wid = lax.axis_index("s") * nc + lax.axis_index("c")
        base = wid * b_per_w

        def body(i, carry):
            off = base + i * _CHUNK
            pltpu.sync_copy(idx_hbm.at[pl.ds(off, _CHUNK)], idx_v)
            pltpu.async_copy(tbl_hbm.at[idx_v], rows_v, sem).wait()
            pltpu.sync_copy(rows_v, out_hbm.at[pl.ds(off, _CHUNK)])
            return carry

        lax.fori_loop(0, n_chunks, body, 0)

    return gather


def kernel(x, table):
    B = x.size
    V, D = table.shape
    scaled = _scale_table(table)
    flat_idx = x.reshape(B).astype(jnp.int32)
    out = _make_gather(B, V, D)(flat_idx, scaled)
    return out.reshape(*x.shape, D)


# speedup vs baseline: 1.1199x; 1.1199x over previous
import functools
import math

import jax
import jax.numpy as jnp
from jax import lax
from jax.experimental import pallas as pl
from jax.experimental.pallas import tpu as pltpu
from jax.experimental.pallas import tpu_sc as plsc

_SCALE = math.sqrt(512.0)
_CHUNK = 32


def _scale_body(t_ref, o_ref):
    o_ref[...] = t_ref[...] * _SCALE


def _scale_table(table):
    return pl.pallas_call(
        _scale_body,
        out_shape=jax.ShapeDtypeStruct(table.shape, table.dtype),
    )(table)


@functools.cache
def _make_gather(B, V, D):
    info = plsc.get_sparse_core_info()
    nc, ns = info.num_cores, info.num_subcores
    nw = nc * ns
    assert B % (nw * 2 * _CHUNK) == 0
    b_per_w = B // nw
    n_chunks = b_per_w // _CHUNK
    n_pairs = n_chunks // 2
    mesh = plsc.VectorSubcoreMesh(core_axis_name="c", subcore_axis_name="s")

    @functools.partial(
        pl.kernel,
        out_type=jax.ShapeDtypeStruct((B, D), jnp.float32),
        mesh=mesh,
        scratch_types=[
            pltpu.VMEM_SHARED((V, D), jnp.float32),
            pltpu.VMEM((_CHUNK,), jnp.int32),
            pltpu.VMEM((_CHUNK,), jnp.int32),
            pltpu.VMEM((_CHUNK, D), jnp.float32),
            pltpu.VMEM((_CHUNK, D), jnp.float32),
            pltpu.SemaphoreType.DMA,
            pltpu.SemaphoreType.DMA,
            pltpu.SemaphoreType.DMA,
            pltpu.SemaphoreType.DMA,
        ],
        compiler_params=pltpu.CompilerParams(use_tc_tiling_on_sc=False),
    )
    def gather(idx_hbm, tbl_hbm, out_hbm, tbl_sh, idx0, idx1, r0, r1,
               g0, g1, o0, o1):
        cid = lax.axis_index("c")
        sid = lax.axis_index("s")
        wid = sid * nc + cid
        base = wid * b_per_w
        idx = (idx0, idx1)
        rows = (r0, r1)
        gsem = (g0, g1)
        osem = (o0, o1)

        @pl.when(sid == 0)
        def _():
            pltpu.sync_copy(tbl_hbm, tbl_sh)

        plsc.subcore_barrier()

        def start_gather(i, b):
            pltpu.sync_copy(idx_hbm.at[pl.ds(base + i * _CHUNK, _CHUNK)],
                            idx[b])
            pltpu.async_copy(tbl_sh.at[idx[b]], rows[b], gsem[b])

        def wait_gather(b):
            pltpu.make_async_copy(tbl_sh.at[idx[b]], rows[b], gsem[b]).wait()

        def start_out(i, b):
            pltpu.async_copy(rows[b],
                             out_hbm.at[pl.ds(base + i * _CHUNK, _CHUNK)],
                             osem[b])

        def wait_out(i, b):
            pltpu.make_async_copy(rows[b],
                                  out_hbm.at[pl.ds(base + i * _CHUNK, _CHUNK)],
                                  osem[b]).wait()

        start_gather(0, 0)
        start_gather(1, 1)

        def pair_body(j, carry):
            for b in (0, 1):
                wait_gather(b)
                start_out(2 * j + b, b)
            for b in (0, 1):
                wait_out(2 * j + b, b)
                start_gather(2 * j + b + 2, b)
            return carry

        lax.fori_loop(0, n_pairs - 1, pair_body, 0, unroll=False)

        last = 2 * (n_pairs - 1)
        for b in (0, 1):
            wait_gather(b)
            start_out(last + b, b)
        for b in (0, 1):
            wait_out(last + b, b)

    return gather


def kernel(x, table):
    B = x.size
    V, D = table.shape
    scaled = _scale_table(table)
    flat_idx = x.reshape(B).astype(jnp.int32)
    out = _make_gather(B, V, D)(flat_idx, scaled)
    return out.reshape(*x.shape, D)
